# baseline (speedup 1.0000x reference)
import functools

import jax
import jax.numpy as jnp
from jax import lax
from jax.experimental import pallas as pl
from jax.experimental.pallas import tpu as pltpu
from jax.experimental.pallas import tpu_sc as plsc

N = 100000
E = 3200000
NC = 2
NS = 16
NW = NC * NS
BATCH = 128
K = 4
GROUP = K * BATCH
U = 2
NGRP = 196
EPW = NGRP * GROUP
EPAD = NW * EPW
NP = 100096
RPT = NP // NS

_mesh = plsc.VectorSubcoreMesh(
    core_axis_name="c", subcore_axis_name="s", num_cores=NC, num_subcores=NS
)
_sc_params = pltpu.CompilerParams(use_tc_tiling_on_sc=False)


def _deg_body(dst3, ones, zeros, out, didx, rows, acc, ssems):
    c = lax.axis_index("c")
    s = lax.axis_index("s")
    w = c * NS + s
    pltpu.sync_copy(zeros.at[pl.ds(s * RPT, RPT)], acc.at[pl.ds(s * RPT, RPT)])
    pltpu.sync_copy(ones, rows)
    plsc.subcore_barrier()

    @pl.loop(0, NGRP // U)
    def _grp(t):
        cps = []
        for u in range(U):
            pltpu.sync_copy(dst3.at[w, pl.ds((t * U + u) * K, K)], didx.at[u])
            for j in range(K):
                cps.append(pltpu.async_copy(
                    rows.at[pl.ds(j * BATCH, BATCH)], acc.at[didx.at[u, j]],
                    ssems[u], add=True,
                ))
        for cp in cps:
            cp.wait()

    plsc.subcore_barrier()
    pltpu.sync_copy(acc.at[pl.ds(s * RPT, RPT)], out.at[c, pl.ds(s * RPT, RPT)])


def _scat_body(width, table, src3, dst3, zeros, out, sidx, didx, rows, acc,
               gsems, ssems):
    c = lax.axis_index("c")
    s = lax.axis_index("s")
    w = c * NS + s
    pltpu.sync_copy(zeros.at[pl.ds(s * RPT, RPT)], acc.at[pl.ds(s * RPT, RPT)])
    plsc.subcore_barrier()

    @pl.loop(0, NGRP // U)
    def _grp(t):
        gcps = []
        for u in range(U):
            g = t * U + u
            pltpu.sync_copy(src3.at[w, pl.ds(g * K, K)], sidx.at[u])
            pltpu.sync_copy(dst3.at[w, pl.ds(g * K, K)], didx.at[u])
            gcps.append([
                pltpu.async_copy(
                    table.at[sidx.at[u, j]],
                    rows.at[u, pl.ds(j * BATCH, BATCH)], gsems[u],
                )
                for j in range(K)
            ])
        scps = []
        for u in range(U):
            for cp in gcps[u]:
                cp.wait()
            for j in range(K):
                scps.append(pltpu.async_copy(
                    rows.at[u, pl.ds(j * BATCH, BATCH)], acc.at[didx.at[u, j]],
                    ssems[u], add=True,
                ))
        for cp in scps:
            cp.wait()

    plsc.subcore_barrier()
    pltpu.sync_copy(acc.at[pl.ds(s * RPT, RPT)], out.at[c, pl.ds(s * RPT, RPT)])


def _deg_pass(dst3, ones, zeros):
    return pl.kernel(
        _deg_body,
        out_type=jax.ShapeDtypeStruct((NC, NP, 1), jnp.float32),
        mesh=_mesh,
        compiler_params=_sc_params,
        scratch_types=[
            pltpu.VMEM((U, K, BATCH), jnp.int32),
            pltpu.VMEM((GROUP, 1), jnp.float32),
            pltpu.VMEM_SHARED((NP, 1), jnp.float32),
            [pltpu.SemaphoreType.DMA] * U,
        ],
    )(dst3, ones, zeros)


def _scatter_pass(width, table, src3, dst3, zeros):
    return pl.kernel(
        functools.partial(_scat_body, width),
        out_type=jax.ShapeDtypeStruct((NC, NP, width), jnp.float32),
        mesh=_mesh,
        compiler_params=_sc_params,
        scratch_types=[
            pltpu.VMEM((U, K, BATCH), jnp.int32),
            pltpu.VMEM((U, K, BATCH), jnp.int32),
            pltpu.VMEM((U, GROUP, width), jnp.float32),
            pltpu.VMEM_SHARED((NP, width), jnp.float32),
            [pltpu.SemaphoreType.DMA] * U,
            [pltpu.SemaphoreType.DMA] * U,
        ],
    )(table, src3, dst3, zeros)


BLK = 2048
GRID = -(-N // BLK)


def _row_spec(width):
    return pl.BlockSpec((BLK, width), lambda i: (i, 0))


def _full_spec(shape):
    return pl.BlockSpec(shape, lambda i: tuple(0 for _ in shape))


def _stage0_body(dpa, dpb, x, dis_o, inv_o, g0_o):
    deg = dpa[...] + dpb[...] + 1.0
    inv = 1.0 / deg
    dis = lax.rsqrt(deg)
    dis_o[...] = dis
    inv_o[...] = inv
    g0_o[...] = x[...] * dis


def _stage0(dpa, dpb, x):
    return pl.pallas_call(
        _stage0_body,
        grid=(GRID,),
        in_specs=[_row_spec(1), _row_spec(1), _row_spec(3)],
        out_specs=[_row_spec(1), _row_spec(1), _row_spec(3)],
        out_shape=[
            jax.ShapeDtypeStruct((N, 1), jnp.float32),
            jax.ShapeDtypeStruct((N, 1), jnp.float32),
            jax.ShapeDtypeStruct((N, 3), jnp.float32),
        ],
    )(dpa, dpb, x)


def _stage1_body(sa, sb, x, dis, inv, w0, b0, h1_o, g1_o):
    s3 = (sa[...] + sb[...])[:, :3]
    pre = dis[...] * s3 + x[...] * inv[...]
    h = jnp.dot(pre, w0[...], preferred_element_type=jnp.float32) + b0[...]
    h = jnp.maximum(h, 0.0)
    h1_o[...] = h
    g1_o[...] = h * dis[...]


def _stage1(sa, sb, x, dis, inv, w0, b0):
    return pl.pallas_call(
        _stage1_body,
        grid=(GRID,),
        in_specs=[
            _row_spec(4), _row_spec(4), _row_spec(3), _row_spec(1),
            _row_spec(1), _full_spec((3, 16)), _full_spec((1, 16)),
        ],
        out_specs=[_row_spec(16), _row_spec(16)],
        out_shape=[
            jax.ShapeDtypeStruct((N, 16), jnp.float32),
            jax.ShapeDtypeStruct((N, 16), jnp.float32),
        ],
    )(sa, sb, x, dis, inv, w0, b0)


def _stage2_body(sa, sb, h1, dis, inv, w1, b1, w2, z2_o, g2_o):
    pre = dis[...] * (sa[...] + sb[...]) + h1[...] * inv[...]
    h2 = jnp.dot(pre, w1[...], preferred_element_type=jnp.float32) + b1[...]
    h2 = jnp.maximum(h2, 0.0)
    z2 = jnp.dot(h2, w2[...], preferred_element_type=jnp.float32)
    z2_o[...] = z2
    g2_o[...] = z2 * dis[...]


def _stage2(sa, sb, h1, dis, inv, w1, b1, w2):
    return pl.pallas_call(
        _stage2_body,
        grid=(GRID,),
        in_specs=[
            _row_spec(16), _row_spec(16), _row_spec(16), _row_spec(1),
            _row_spec(1), _full_spec((16, 16)), _full_spec((1, 16)),
            _full_spec((16, 1)),
        ],
        out_specs=[_row_spec(1), _row_spec(1)],
        out_shape=[
            jax.ShapeDtypeStruct((N, 1), jnp.float32),
            jax.ShapeDtypeStruct((N, 1), jnp.float32),
        ],
    )(sa, sb, h1, dis, inv, w1, b1, w2)


def _stage3_body(sa, sb, z2, dis, inv, b2, y_o):
    pre = dis[...] * (sa[...] + sb[...]) + z2[...] * inv[...] + b2[...]
    y_o[...] = jax.nn.sigmoid(pre)


def _stage3(sa, sb, z2, dis, inv, b2):
    return pl.pallas_call(
        _stage3_body,
        grid=(GRID,),
        in_specs=[
            _row_spec(1), _row_spec(1), _row_spec(1), _row_spec(1),
            _row_spec(1), _full_spec((1, 1)),
        ],
        out_specs=_row_spec(1),
        out_shape=jax.ShapeDtypeStruct((N, 1), jnp.float32),
    )(sa, sb, z2, dis, inv, b2)


def kernel(x, edge_index, W0, b0, W1, b1, W2, b2):
    pad = EPAD - E
    src = jnp.concatenate([edge_index[0], jnp.zeros((pad,), jnp.int32)])
    dst = jnp.concatenate(
        [edge_index[1], jnp.full((pad,), N, jnp.int32)]
    )
    src3 = src.reshape(NW, NGRP * K, BATCH)
    dst3 = dst.reshape(NW, NGRP * K, BATCH)

    z1 = jnp.zeros((NP, 1), jnp.float32)
    z4 = jnp.zeros((NP, 4), jnp.float32)
    z16 = jnp.zeros((NP, 16), jnp.float32)
    ones = jnp.ones((GROUP, 1), jnp.float32)

    degp = _deg_pass(dst3, ones, z1)
    dis, inv, g0 = _stage0(degp[0, :N], degp[1, :N], x)
    g0p = jnp.pad(g0, ((0, 0), (0, 1)))

    s0 = _scatter_pass(4, g0p, src3, dst3, z4)
    h1, g1 = _stage1(
        s0[0, :N], s0[1, :N], x, dis, inv, W0, b0.reshape(1, 16)
    )

    s1 = _scatter_pass(16, g1, src3, dst3, z16)
    z2, g2 = _stage2(
        s1[0, :N], s1[1, :N], h1, dis, inv, W1, b1.reshape(1, 16),
        W2,
    )

    s2 = _scatter_pass(1, g2, src3, dst3, z1)
    return _stage3(s2[0, :N], s2[1, :N], z2, dis, inv, b2.reshape(1, 1))


# speedup vs baseline: 1.1007x; 1.1007x over previous
import functools

import jax
import jax.numpy as jnp
from jax import lax
from jax.experimental import pallas as pl
from jax.experimental.pallas import tpu as pltpu
from jax.experimental.pallas import tpu_sc as plsc

N = 100000
E = 3200000
NC = 2
NS = 16
NW = NC * NS
BATCH = 128
K = 4
GROUP = K * BATCH
U = 2
NGRP = 196
EPW = NGRP * GROUP
EPAD = NW * EPW
NP = 100096
RPT = NP // NS

_mesh = plsc.VectorSubcoreMesh(
    core_axis_name="c", subcore_axis_name="s", num_cores=NC, num_subcores=NS
)
_sc_params = pltpu.CompilerParams(use_tc_tiling_on_sc=False)


def _deg_body(dst3, ones, zeros, out, didx, rows, acc, ssems):
    c = lax.axis_index("c")
    s = lax.axis_index("s")
    w = c * NS + s
    pltpu.sync_copy(zeros.at[pl.ds(s * RPT, RPT)], acc.at[pl.ds(s * RPT, RPT)])
    pltpu.sync_copy(ones, rows)
    plsc.subcore_barrier()

    @pl.loop(0, NGRP // U)
    def _grp(t):
        cps = []
        for u in range(U):
            pltpu.sync_copy(dst3.at[w, pl.ds((t * U + u) * K, K)], didx.at[u])
            for j in range(K):
                cps.append(pltpu.async_copy(
                    rows.at[pl.ds(j * BATCH, BATCH)], acc.at[didx.at[u, j]],
                    ssems[u], add=True,
                ))
        for cp in cps:
            cp.wait()

    plsc.subcore_barrier()
    pltpu.sync_copy(acc.at[pl.ds(s * RPT, RPT)], out.at[c, pl.ds(s * RPT, RPT)])


def _scat_body(width, table, src3, dst3, zeros, out, sidx, didx, rows, acc,
               gsems, ssems):
    c = lax.axis_index("c")
    s = lax.axis_index("s")
    w = c * NS + s
    pltpu.sync_copy(zeros.at[pl.ds(s * RPT, RPT)], acc.at[pl.ds(s * RPT, RPT)])
    plsc.subcore_barrier()

    @pl.loop(0, NGRP // U)
    def _grp(t):
        gcps = []
        for u in range(U):
            g = t * U + u
            pltpu.sync_copy(src3.at[w, pl.ds(g * K, K)], sidx.at[u])
            pltpu.sync_copy(dst3.at[w, pl.ds(g * K, K)], didx.at[u])
            gcps.append([
                pltpu.async_copy(
                    table.at[sidx.at[u, j]],
                    rows.at[u, pl.ds(j * BATCH, BATCH)], gsems[u],
                )
                for j in range(K)
            ])
        scps = []
        for u in range(U):
            for cp in gcps[u]:
                cp.wait()
            for j in range(K):
                scps.append(pltpu.async_copy(
                    rows.at[u, pl.ds(j * BATCH, BATCH)], acc.at[didx.at[u, j]],
                    ssems[u], add=True,
                ))
        for cp in scps:
            cp.wait()

    plsc.subcore_barrier()
    pltpu.sync_copy(acc.at[pl.ds(s * RPT, RPT)], out.at[c, pl.ds(s * RPT, RPT)])


def _deg_pass(dst3, ones, zeros):
    return pl.kernel(
        _deg_body,
        out_type=jax.ShapeDtypeStruct((NC, NP, 1), jnp.float32),
        mesh=_mesh,
        compiler_params=_sc_params,
        scratch_types=[
            pltpu.VMEM((U, K, BATCH), jnp.int32),
            pltpu.VMEM((GROUP, 1), jnp.float32),
            pltpu.VMEM_SHARED((NP, 1), jnp.float32),
            [pltpu.SemaphoreType.DMA] * U,
        ],
    )(dst3, ones, zeros)


def _scatter_pass(width, table, src3, dst3, zeros):
    return pl.kernel(
        functools.partial(_scat_body, width),
        out_type=jax.ShapeDtypeStruct((NC, NP, width), jnp.float32),
        mesh=_mesh,
        compiler_params=_sc_params,
        scratch_types=[
            pltpu.VMEM((U, K, BATCH), jnp.int32),
            pltpu.VMEM((U, K, BATCH), jnp.int32),
            pltpu.VMEM((U, GROUP, width), jnp.float32),
            pltpu.VMEM_SHARED((NP, width), jnp.float32),
            [pltpu.SemaphoreType.DMA] * U,
            [pltpu.SemaphoreType.DMA] * U,
        ],
    )(table, src3, dst3, zeros)


BLK = 2048
GRID = -(-N // BLK)


def _row_spec(width):
    return pl.BlockSpec((BLK, width), lambda i: (i, 0))


def _full_spec(shape):
    return pl.BlockSpec(shape, lambda i: tuple(0 for _ in shape))


def _stage0_body(dp, x, dis_o, inv_o, g0_o):
    d = dp[...]
    deg = d[0] + d[1] + 1.0
    inv = 1.0 / deg
    dis = lax.rsqrt(deg)
    dis_o[...] = dis
    inv_o[...] = inv
    g0_o[:, :3] = x[...] * dis
    g0_o[:, 3:] = jnp.zeros((BLK, 1), jnp.float32)


def _part_spec(width):
    return pl.BlockSpec((NC, BLK, width), lambda i: (0, i, 0))


def _stage0(dp, x):
    return pl.pallas_call(
        _stage0_body,
        grid=(GRID,),
        in_specs=[_part_spec(1), _row_spec(3)],
        out_specs=[_row_spec(1), _row_spec(1), _row_spec(4)],
        out_shape=[
            jax.ShapeDtypeStruct((N, 1), jnp.float32),
            jax.ShapeDtypeStruct((N, 1), jnp.float32),
            jax.ShapeDtypeStruct((N, 4), jnp.float32),
        ],
    )(dp, x)


def _stage1_body(s0, x, dis, inv, w0, b0, h1_o, g1_o):
    sp = s0[...]
    s3 = (sp[0] + sp[1])[:, :3]
    pre = dis[...] * s3 + x[...] * inv[...]
    h = jnp.dot(pre, w0[...], preferred_element_type=jnp.float32) + b0[...]
    h = jnp.maximum(h, 0.0)
    h1_o[...] = h
    g1_o[...] = h * dis[...]


def _stage1(s0, x, dis, inv, w0, b0):
    return pl.pallas_call(
        _stage1_body,
        grid=(GRID,),
        in_specs=[
            _part_spec(4), _row_spec(3), _row_spec(1),
            _row_spec(1), _full_spec((3, 16)), _full_spec((1, 16)),
        ],
        out_specs=[_row_spec(16), _row_spec(16)],
        out_shape=[
            jax.ShapeDtypeStruct((N, 16), jnp.float32),
            jax.ShapeDtypeStruct((N, 16), jnp.float32),
        ],
    )(s0, x, dis, inv, w0, b0)


def _stage2_body(s1, h1, dis, inv, w1, b1, w2, z2_o, g2_o):
    sp = s1[...]
    pre = dis[...] * (sp[0] + sp[1]) + h1[...] * inv[...]
    h2 = jnp.dot(pre, w1[...], preferred_element_type=jnp.float32) + b1[...]
    h2 = jnp.maximum(h2, 0.0)
    z2 = jnp.dot(h2, w2[...], preferred_element_type=jnp.float32)
    z2_o[...] = z2
    g2_o[...] = z2 * dis[...]


def _stage2(s1, h1, dis, inv, w1, b1, w2):
    return pl.pallas_call(
        _stage2_body,
        grid=(GRID,),
        in_specs=[
            _part_spec(16), _row_spec(16), _row_spec(1),
            _row_spec(1), _full_spec((16, 16)), _full_spec((1, 16)),
            _full_spec((16, 1)),
        ],
        out_specs=[_row_spec(1), _row_spec(1)],
        out_shape=[
            jax.ShapeDtypeStruct((N, 1), jnp.float32),
            jax.ShapeDtypeStruct((N, 1), jnp.float32),
        ],
    )(s1, h1, dis, inv, w1, b1, w2)


def _stage3_body(s2, z2, dis, inv, b2, y_o):
    sp = s2[...]
    pre = dis[...] * (sp[0] + sp[1]) + z2[...] * inv[...] + b2[...]
    y_o[...] = jax.nn.sigmoid(pre)


def _stage3(s2, z2, dis, inv, b2):
    return pl.pallas_call(
        _stage3_body,
        grid=(GRID,),
        in_specs=[
            _part_spec(1), _row_spec(1), _row_spec(1),
            _row_spec(1), _full_spec((1, 1)),
        ],
        out_specs=_row_spec(1),
        out_shape=jax.ShapeDtypeStruct((N, 1), jnp.float32),
    )(s2, z2, dis, inv, b2)


def kernel(x, edge_index, W0, b0, W1, b1, W2, b2):
    pad = EPAD - E
    src = jnp.concatenate([edge_index[0], jnp.zeros((pad,), jnp.int32)])
    dst = jnp.concatenate(
        [edge_index[1], jnp.full((pad,), N, jnp.int32)]
    )
    src3 = src.reshape(NW, NGRP * K, BATCH)
    dst3 = dst.reshape(NW, NGRP * K, BATCH)

    z1 = jnp.zeros((NP, 1), jnp.float32)
    z4 = jnp.zeros((NP, 4), jnp.float32)
    z16 = jnp.zeros((NP, 16), jnp.float32)
    ones = jnp.ones((GROUP, 1), jnp.float32)

    degp = _deg_pass(dst3, ones, z1)
    dis, inv, g0 = _stage0(degp, x)

    s0 = _scatter_pass(4, g0, src3, dst3, z4)
    h1, g1 = _stage1(s0, x, dis, inv, W0, b0.reshape(1, 16))

    s1 = _scatter_pass(16, g1, src3, dst3, z16)
    z2, g2 = _stage2(s1, h1, dis, inv, W1, b1.reshape(1, 16), W2)

    s2 = _scatter_pass(1, g2, src3, dst3, z1)
    return _stage3(s2, z2, dis, inv, b2.reshape(1, 1))
